# initial kernel scaffold (unmeasured)
import jax
import jax.numpy as jnp
from jax import lax
from jax.experimental import pallas as pl
from jax.experimental.pallas import tpu as pltpu

N_DEV = 4


def kernel(x, w_mat, scale_x, scale_w):
    m_per, k = x.shape
    _, n = w_mat.shape
    n_per = n // N_DEV
    m_tot = m_per * N_DEV

    def body(x_ref, w_ref, sx_ref, sw_ref, out_ref,
             xb_ref, wb_ref, comm_ref, send_sems, recv_sems):
        my = lax.axis_index("i")

        barrier_sem = pltpu.get_barrier_semaphore()
        for h in range(1, N_DEV):
            pl.semaphore_signal(
                barrier_sem, inc=1,
                device_id=((my + h) % N_DEV,),
                device_id_type=pl.DeviceIdType.MESH,
            )
        pl.semaphore_wait(barrier_sem, N_DEV - 1)

        scale = sx_ref[0] * sw_ref[0]
        xb_ref[...] = x_ref[...].astype(jnp.bfloat16)
        wb_ref[...] = w_ref[...].astype(jnp.bfloat16)

        sends = []
        for h in range(1, N_DEV):
            j = (my + h) % N_DEV
            blk = jnp.dot(
                xb_ref[...],
                wb_ref[:, pl.ds(j * n_per, n_per)],
                preferred_element_type=jnp.float32,
            ) * scale
            comm_ref[h - 1, :, :] = blk
            rdma = pltpu.make_async_remote_copy(
                src_ref=comm_ref.at[h - 1],
                dst_ref=out_ref.at[pl.ds(my * m_per, m_per), :],
                send_sem=send_sems.at[h - 1],
                recv_sem=recv_sems.at[h - 1],
                device_id=(j,),
                device_id_type=pl.DeviceIdType.MESH,
            )
            rdma.start()
            sends.append(rdma)

        blk = jnp.dot(
            xb_ref[...],
            wb_ref[:, pl.ds(my * n_per, n_per)],
            preferred_element_type=jnp.float32,
        ) * scale
        out_ref[pl.ds(my * m_per, m_per), :] = blk

        for h in range(1, N_DEV):
            s = (my - h) % N_DEV
            recv = pltpu.make_async_remote_copy(
                src_ref=comm_ref.at[h - 1],
                dst_ref=out_ref.at[pl.ds(s * m_per, m_per), :],
                send_sem=send_sems.at[h - 1],
                recv_sem=recv_sems.at[h - 1],
                device_id=(s,),
                device_id_type=pl.DeviceIdType.MESH,
            )
            recv.wait_recv()

        for rdma in sends:
            rdma.wait_send()

    return pl.pallas_call(
        body,
        out_shape=jax.ShapeDtypeStruct((m_tot, n_per), jnp.float32),
        in_specs=[
            pl.BlockSpec(memory_space=pltpu.VMEM),
            pl.BlockSpec(memory_space=pltpu.VMEM),
            pl.BlockSpec(memory_space=pltpu.SMEM),
            pl.BlockSpec(memory_space=pltpu.SMEM),
        ],
        out_specs=pl.BlockSpec(memory_space=pltpu.VMEM),
        scratch_shapes=[
            pltpu.VMEM((m_per, k), jnp.bfloat16),
            pltpu.VMEM((k, n), jnp.bfloat16),
            pltpu.VMEM((N_DEV - 1, m_per, n_per), jnp.float32),
            pltpu.SemaphoreType.DMA((N_DEV - 1,)),
            pltpu.SemaphoreType.DMA((N_DEV - 1,)),
        ],
        compiler_params=pltpu.CompilerParams(collective_id=0),
    )(x, w_mat, scale_x, scale_w)


# baseline (device time: 92544 ns/iter reference)
import jax
import jax.numpy as jnp
from jax import lax
from jax.experimental import pallas as pl
from jax.experimental.pallas import tpu as pltpu

N_DEV = 4


def kernel(x, w_mat, scale_x, scale_w):
    m_per, k = x.shape
    _, n = w_mat.shape
    n_per = n // N_DEV
    m_tot = m_per * N_DEV

    def body(x_ref, w_ref, sx_ref, sw_ref, out_ref,
             comm_ref, send_sems, recv_sems):
        my = lax.axis_index("i")

        barrier_sem = pltpu.get_barrier_semaphore()
        for h in range(1, N_DEV):
            pl.semaphore_signal(
                barrier_sem, inc=1,
                device_id=((my + h) % N_DEV,),
                device_id_type=pl.DeviceIdType.MESH,
            )
        pl.semaphore_wait(barrier_sem, N_DEV - 1)

        scale = sx_ref[0] * sw_ref[0]

        sends = []
        for h in range(1, N_DEV):
            j = (my + h) % N_DEV
            blk = jnp.dot(
                x_ref[...],
                w_ref[:, pl.ds(j * n_per, n_per)],
                preferred_element_type=jnp.float32,
            ) * scale
            comm_ref[h - 1, :, :] = blk
            rdma = pltpu.make_async_remote_copy(
                src_ref=comm_ref.at[h - 1],
                dst_ref=out_ref.at[pl.ds(my * m_per, m_per), :],
                send_sem=send_sems.at[h - 1],
                recv_sem=recv_sems.at[h - 1],
                device_id=(j,),
                device_id_type=pl.DeviceIdType.MESH,
            )
            rdma.start()
            sends.append(rdma)

        blk = jnp.dot(
            x_ref[...],
            w_ref[:, pl.ds(my * n_per, n_per)],
            preferred_element_type=jnp.float32,
        ) * scale
        out_ref[pl.ds(my * m_per, m_per), :] = blk

        for h in range(1, N_DEV):
            s = (my - h) % N_DEV
            recv = pltpu.make_async_remote_copy(
                src_ref=comm_ref.at[h - 1],
                dst_ref=out_ref.at[pl.ds(s * m_per, m_per), :],
                send_sem=send_sems.at[h - 1],
                recv_sem=recv_sems.at[h - 1],
                device_id=(s,),
                device_id_type=pl.DeviceIdType.MESH,
            )
            recv.wait_recv()

        for rdma in sends:
            rdma.wait_send()

    return pl.pallas_call(
        body,
        out_shape=jax.ShapeDtypeStruct((m_tot, n_per), jnp.float32),
        in_specs=[
            pl.BlockSpec(memory_space=pltpu.VMEM),
            pl.BlockSpec(memory_space=pltpu.VMEM),
            pl.BlockSpec(memory_space=pltpu.SMEM),
            pl.BlockSpec(memory_space=pltpu.SMEM),
        ],
        out_specs=pl.BlockSpec(memory_space=pltpu.VMEM),
        scratch_shapes=[
            pltpu.VMEM((N_DEV - 1, m_per, n_per), jnp.float32),
            pltpu.SemaphoreType.DMA((N_DEV - 1,)),
            pltpu.SemaphoreType.DMA((N_DEV - 1,)),
        ],
        compiler_params=pltpu.CompilerParams(collective_id=0),
    )(x.astype(jnp.bfloat16), w_mat.astype(jnp.bfloat16), scale_x, scale_w)


# device time: 47120 ns/iter; 1.9640x vs baseline; 1.9640x over previous
import jax
import jax.numpy as jnp
from jax import lax
from jax.experimental import pallas as pl
from jax.experimental.pallas import tpu as pltpu

N_DEV = 4
XC = 4
WC = 2


def kernel(x, w_mat, scale_x, scale_w):
    m_per, k = x.shape
    _, n = w_mat.shape
    n_per = n // N_DEV
    m_tot = m_per * N_DEV
    mx = m_per // XC
    kw = k // WC

    def body(x_hbm, w_hbm, sx_ref, sw_ref, out_ref,
             xstage, wstage, x8, w8, send_buf, recv_buf,
             xsems, wsems, send_sems, recv_sems):
        my = lax.axis_index("i")

        barrier_sem = pltpu.get_barrier_semaphore()
        for h in range(1, N_DEV):
            pl.semaphore_signal(
                barrier_sem, inc=1,
                device_id=((my + h) % N_DEV,),
                device_id_type=pl.DeviceIdType.MESH,
            )
        pl.semaphore_wait(barrier_sem, N_DEV - 1)

        scale = sx_ref[0] * sw_ref[0]

        def start_x(c):
            cp = pltpu.make_async_copy(
                x_hbm.at[pl.ds(c * mx, mx), :],
                xstage.at[c % 2],
                xsems.at[c % 2],
            )
            cp.start()
            return cp

        def start_w(i):
            b, c = divmod(i, WC)
            j = (my + 1 + b) % N_DEV
            cp = pltpu.make_async_copy(
                w_hbm.at[pl.ds(c * kw, kw), pl.ds(j * n_per, n_per)],
                wstage.at[i % 2],
                wsems.at[i % 2],
            )
            cp.start()
            return cp

        xcopies = {0: start_x(0), 1: start_x(1)}
        wcopies = {0: start_w(0), 1: start_w(1)}

        for c in range(XC):
            xcopies[c].wait()
            x8[pl.ds(c * mx, mx), :] = xstage[c % 2, :, :].astype(
                jnp.float8_e4m3fn)
            if c + 2 < XC:
                xcopies[c + 2] = start_x(c + 2)

        sends = []
        for b in range(N_DEV):
            j = (my + 1 + b) % N_DEV
            for c in range(WC):
                i = b * WC + c
                wcopies[i].wait()
                w8[b, pl.ds(c * kw, kw), :] = wstage[i % 2, :, :].astype(
                    jnp.float8_e5m2)
                if i + 2 < N_DEV * WC:
                    wcopies[i + 2] = start_w(i + 2)

            blk = jnp.dot(x8[...], w8[b, :, :],
                          preferred_element_type=jnp.float32)
            if b < N_DEV - 1:
                send_buf[b, :, :] = blk.astype(jnp.bfloat16)
                rdma = pltpu.make_async_remote_copy(
                    src_ref=send_buf.at[b],
                    dst_ref=recv_buf.at[b],
                    send_sem=send_sems.at[b],
                    recv_sem=recv_sems.at[b],
                    device_id=(j,),
                    device_id_type=pl.DeviceIdType.MESH,
                )
                rdma.start()
                sends.append(rdma)
            else:
                out_ref[pl.ds(my * m_per, m_per), :] = blk * scale

        for b in range(N_DEV - 1):
            s = (my - 1 - b) % N_DEV
            recv = pltpu.make_async_remote_copy(
                src_ref=send_buf.at[b],
                dst_ref=recv_buf.at[b],
                send_sem=send_sems.at[b],
                recv_sem=recv_sems.at[b],
                device_id=(s,),
                device_id_type=pl.DeviceIdType.MESH,
            )
            recv.wait_recv()
            out_ref[pl.ds(s * m_per, m_per), :] = (
                recv_buf[b, :, :].astype(jnp.float32) * scale)

        for rdma in sends:
            rdma.wait_send()

    return pl.pallas_call(
        body,
        out_shape=jax.ShapeDtypeStruct((m_tot, n_per), jnp.float32),
        in_specs=[
            pl.BlockSpec(memory_space=pl.ANY),
            pl.BlockSpec(memory_space=pl.ANY),
            pl.BlockSpec(memory_space=pltpu.SMEM),
            pl.BlockSpec(memory_space=pltpu.SMEM),
        ],
        out_specs=pl.BlockSpec(memory_space=pltpu.VMEM),
        scratch_shapes=[
            pltpu.VMEM((2, mx, k), jnp.float32),
            pltpu.VMEM((2, kw, n_per), jnp.float32),
            pltpu.VMEM((m_per, k), jnp.float8_e4m3fn),
            pltpu.VMEM((N_DEV, k, n_per), jnp.float8_e5m2),
            pltpu.VMEM((N_DEV - 1, m_per, n_per), jnp.bfloat16),
            pltpu.VMEM((N_DEV - 1, m_per, n_per), jnp.bfloat16),
            pltpu.SemaphoreType.DMA((2,)),
            pltpu.SemaphoreType.DMA((2,)),
            pltpu.SemaphoreType.DMA((N_DEV - 1,)),
            pltpu.SemaphoreType.DMA((N_DEV - 1,)),
        ],
        compiler_params=pltpu.CompilerParams(collective_id=0),
    )(x, w_mat, scale_x, scale_w)
